# baseline (speedup 1.0000x reference)
import functools

import jax
import jax.numpy as jnp
from jax import lax
from jax.experimental import pallas as pl
from jax.experimental.pallas import tpu as pltpu
from jax.experimental.pallas import tpu_sc as plsc

T, D, F, E, K = 2048, 768, 2048, 8, 2
TM = 128
NSTEPS = 40
P = NSTEPS * TM

NC, NS = 2, 16
NW = NC * NS
TW = T // NW

_sc_mesh = plsc.VectorSubcoreMesh(core_axis_name="c", subcore_axis_name="s")


def _routing_tables(router_logits):
    eids = jnp.arange(E, dtype=jnp.int32)
    m1 = jnp.max(router_logits, axis=-1)
    a1 = jnp.argmax(router_logits, axis=-1).astype(jnp.int32)
    masked = jnp.where(eids[None, :] == a1[:, None], -jnp.inf, router_logits)
    m2 = jnp.max(masked, axis=-1)
    a2 = jnp.argmax(masked, axis=-1).astype(jnp.int32)
    w1 = jax.nn.sigmoid(m1 - m2)
    flat_e = jnp.stack([a1, a2], axis=1).reshape(-1)
    flat_w = jnp.stack([w1, 1.0 - w1], axis=1).reshape(-1)
    S = T * K
    onehot = (flat_e[:, None] == eids[None, :]).astype(jnp.int32)
    csum = jnp.cumsum(onehot, axis=0)
    counts = csum[-1]
    rank = jnp.sum(csum * onehot, axis=1) - 1
    tiles_e = (counts + TM - 1) // TM
    tile_cum = jnp.cumsum(tiles_e)
    pad_off = (tile_cum - tiles_e) * TM
    ppos = pad_off[flat_e] + rank
    step_ids = jnp.arange(NSTEPS, dtype=jnp.int32)
    step_expert = jnp.sum(
        (tile_cum[None, :] <= step_ids[:, None]).astype(jnp.int32), axis=1)
    last_used = jnp.max(jnp.where(counts > 0, jnp.arange(E, dtype=jnp.int32), 0))
    step_expert = jnp.minimum(step_expert, last_used)
    active = (step_ids < tile_cum[-1]).astype(jnp.int32)
    pp = ppos.reshape(T, K)
    return (step_expert, active, w1, 1.0 - w1,
            pp[:, 0].astype(jnp.int32), pp[:, 1].astype(jnp.int32))


@functools.partial(
    pl.kernel,
    mesh=_sc_mesh,
    out_type=jax.ShapeDtypeStruct((P, D), jnp.float32),
    scratch_types=[
        pltpu.VMEM((TW,), jnp.int32),
        pltpu.VMEM((TW,), jnp.int32),
        pltpu.VMEM((TW, D), jnp.float32),
        pltpu.SemaphoreType.DMA,
    ],
)
def _sc_dispatch(x_hbm, p0_hbm, p1_hbm, xs_hbm, i0, i1, buf, sem):
    wid = lax.axis_index("s") * NC + lax.axis_index("c")
    base = wid * TW
    pltpu.sync_copy(p0_hbm.at[pl.ds(base, TW)], i0)
    pltpu.sync_copy(p1_hbm.at[pl.ds(base, TW)], i1)
    pltpu.sync_copy(x_hbm.at[pl.ds(base, TW)], buf)
    s0 = pltpu.async_copy(buf, xs_hbm.at[i0], sem)
    s1 = pltpu.async_copy(buf, xs_hbm.at[i1], sem)
    s0.wait()
    s1.wait()


def _ffn_body(se_ref, act_ref, xs_ref, w1_ref, w3_ref, w2_ref, ys_ref):
    s = pl.program_id(0)

    @pl.when(act_ref[s] > 0)
    def _():
        xt = xs_ref[...]
        g = jax.lax.dot_general(xt, w1_ref[0], (((1,), (1,)), ((), ())),
                                preferred_element_type=jnp.float32)
        u = jax.lax.dot_general(xt, w3_ref[0], (((1,), (1,)), ((), ())),
                                preferred_element_type=jnp.float32)
        h = (g * jax.nn.sigmoid(g)) * u
        ys_ref[...] = jax.lax.dot_general(
            h, w2_ref[0], (((1,), (1,)), ((), ())),
            preferred_element_type=jnp.float32)


def _grouped_ffn(xs, W1, W2, W3, step_expert, active):
    grid_spec = pltpu.PrefetchScalarGridSpec(
        num_scalar_prefetch=2,
        grid=(NSTEPS,),
        in_specs=[
            pl.BlockSpec((TM, D), lambda s, se, act: (s, 0)),
            pl.BlockSpec((1, F, D), lambda s, se, act: (se[s], 0, 0)),
            pl.BlockSpec((1, F, D), lambda s, se, act: (se[s], 0, 0)),
            pl.BlockSpec((1, D, F), lambda s, se, act: (se[s], 0, 0)),
        ],
        out_specs=pl.BlockSpec((TM, D), lambda s, se, act: (s, 0)),
    )
    return pl.pallas_call(
        _ffn_body,
        grid_spec=grid_spec,
        out_shape=jax.ShapeDtypeStruct((P, D), jnp.float32),
        compiler_params=pltpu.CompilerParams(
            dimension_semantics=("arbitrary",)),
    )(step_expert, active, xs, W1, W3, W2)


@functools.partial(
    pl.kernel,
    mesh=_sc_mesh,
    out_type=jax.ShapeDtypeStruct((T, D), jnp.float32),
    scratch_types=[
        pltpu.VMEM((TW,), jnp.int32),
        pltpu.VMEM((TW,), jnp.int32),
        pltpu.VMEM((TW,), jnp.float32),
        pltpu.VMEM((TW,), jnp.float32),
        pltpu.VMEM((TW, D), jnp.float32),
        pltpu.VMEM((TW, D), jnp.float32),
        pltpu.SemaphoreType.DMA,
    ],
)
def _sc_combine(ys_hbm, p0_hbm, p1_hbm, wa_hbm, wb_hbm, out_hbm,
                i0, i1, wv0, wv1, b0, b1, sem):
    wid = lax.axis_index("s") * NC + lax.axis_index("c")
    base = wid * TW
    pltpu.sync_copy(p0_hbm.at[pl.ds(base, TW)], i0)
    pltpu.sync_copy(p1_hbm.at[pl.ds(base, TW)], i1)
    pltpu.sync_copy(wa_hbm.at[pl.ds(base, TW)], wv0)
    pltpu.sync_copy(wb_hbm.at[pl.ds(base, TW)], wv1)
    c0 = pltpu.async_copy(ys_hbm.at[i0], b0, sem)
    c1 = pltpu.async_copy(ys_hbm.at[i1], b1, sem)
    c0.wait()
    c1.wait()

    dnums = lax.GatherDimensionNumbers(
        offset_dims=(), collapsed_slice_dims=(0,), start_index_map=(0,))

    def _bcast(vec, lane):
        idx = jnp.full((16, 1), lane, jnp.int32)
        return lax.gather(vec, idx, dnums, (1,),
                          mode=lax.GatherScatterMode.PROMISE_IN_BOUNDS)

    def _comb_row(r, carry):
        g = r // 16
        lane = r - g * 16
        wa_b = _bcast(wv0[pl.ds(g * 16, 16)], lane)
        wb_b = _bcast(wv1[pl.ds(g * 16, 16)], lane)
        for j in range(D // 16):
            sl = pl.ds(j * 16, 16)
            b0[r, sl] = wa_b * b0[r, sl] + wb_b * b1[r, sl]
        return carry

    lax.fori_loop(0, TW, _comb_row, None)
    pltpu.sync_copy(b0, out_hbm.at[pl.ds(base, TW)])


def kernel(x, router_logits, W1, W2, W3):
    step_expert, active, wa, wb, p0, p1 = _routing_tables(router_logits)
    xs = _sc_dispatch(x, p0, p1)
    ys = _grouped_ffn(xs, W1, W2, W3, step_expert, active)
    return _sc_combine(ys, p0, p1, wa, wb)


# speedup vs baseline: 1.3412x; 1.3412x over previous
import functools

import jax
import jax.numpy as jnp
from jax import lax
from jax.experimental import pallas as pl
from jax.experimental.pallas import tpu as pltpu
from jax.experimental.pallas import tpu_sc as plsc

T, D, F, E, K = 2048, 768, 2048, 8, 2
TM = 256
NSTEPS = 24
P = NSTEPS * TM

NC, NS = 2, 16
NW = NC * NS
TW = T // NW

_sc_mesh = plsc.VectorSubcoreMesh(core_axis_name="c", subcore_axis_name="s")


def _routing_tables(router_logits):
    eids = jnp.arange(E, dtype=jnp.int32)
    m1 = jnp.max(router_logits, axis=-1)
    a1 = jnp.argmax(router_logits, axis=-1).astype(jnp.int32)
    masked = jnp.where(eids[None, :] == a1[:, None], -jnp.inf, router_logits)
    m2 = jnp.max(masked, axis=-1)
    a2 = jnp.argmax(masked, axis=-1).astype(jnp.int32)
    w1 = jax.nn.sigmoid(m1 - m2)
    flat_e = jnp.stack([a1, a2], axis=1).reshape(-1)
    flat_w = jnp.stack([w1, 1.0 - w1], axis=1).reshape(-1)
    S = T * K
    onehot = (flat_e[:, None] == eids[None, :]).astype(jnp.int32)
    csum = jnp.cumsum(onehot, axis=0)
    counts = csum[-1]
    rank = jnp.sum(csum * onehot, axis=1) - 1
    tiles_e = (counts + TM - 1) // TM
    tile_cum = jnp.cumsum(tiles_e)
    pad_off = (tile_cum - tiles_e) * TM
    ppos = pad_off[flat_e] + rank
    step_ids = jnp.arange(NSTEPS, dtype=jnp.int32)
    step_expert = jnp.sum(
        (tile_cum[None, :] <= step_ids[:, None]).astype(jnp.int32), axis=1)
    last_used = jnp.max(jnp.where(counts > 0, jnp.arange(E, dtype=jnp.int32), 0))
    step_expert = jnp.minimum(step_expert, last_used)
    active = (step_ids < tile_cum[-1]).astype(jnp.int32)
    pp = ppos.reshape(T, K)
    return (step_expert, active, w1, 1.0 - w1,
            pp[:, 0].astype(jnp.int32), pp[:, 1].astype(jnp.int32))


@functools.partial(
    pl.kernel,
    mesh=_sc_mesh,
    out_type=jax.ShapeDtypeStruct((P, D), jnp.float32),
    scratch_types=[
        pltpu.VMEM((TW,), jnp.int32),
        pltpu.VMEM((TW,), jnp.int32),
        pltpu.VMEM((TW, D), jnp.float32),
        pltpu.SemaphoreType.DMA,
    ],
)
def _sc_dispatch(x_hbm, p0_hbm, p1_hbm, xs_hbm, i0, i1, buf, sem):
    wid = lax.axis_index("s") * NC + lax.axis_index("c")
    base = wid * TW
    pltpu.sync_copy(p0_hbm.at[pl.ds(base, TW)], i0)
    pltpu.sync_copy(p1_hbm.at[pl.ds(base, TW)], i1)
    pltpu.sync_copy(x_hbm.at[pl.ds(base, TW)], buf)
    s0 = pltpu.async_copy(buf, xs_hbm.at[i0], sem)
    s1 = pltpu.async_copy(buf, xs_hbm.at[i1], sem)
    s0.wait()
    s1.wait()


def _ffn_body(se_ref, act_ref, xs_ref, w1_ref, w3_ref, w2_ref, ys_ref):
    s = pl.program_id(0)

    @pl.when(act_ref[s] > 0)
    def _():
        xt = xs_ref[...]
        g = jax.lax.dot_general(xt, w1_ref[0], (((1,), (1,)), ((), ())),
                                preferred_element_type=jnp.float32)
        u = jax.lax.dot_general(xt, w3_ref[0], (((1,), (1,)), ((), ())),
                                preferred_element_type=jnp.float32)
        h = (g * jax.nn.sigmoid(g)) * u
        ys_ref[...] = jax.lax.dot_general(
            h, w2_ref[0], (((1,), (1,)), ((), ())),
            preferred_element_type=jnp.float32)


def _grouped_ffn(xs, W1, W2, W3, step_expert, active):
    grid_spec = pltpu.PrefetchScalarGridSpec(
        num_scalar_prefetch=2,
        grid=(NSTEPS,),
        in_specs=[
            pl.BlockSpec((TM, D), lambda s, se, act: (s, 0)),
            pl.BlockSpec((1, F, D), lambda s, se, act: (se[s], 0, 0)),
            pl.BlockSpec((1, F, D), lambda s, se, act: (se[s], 0, 0)),
            pl.BlockSpec((1, D, F), lambda s, se, act: (se[s], 0, 0)),
        ],
        out_specs=pl.BlockSpec((TM, D), lambda s, se, act: (s, 0)),
    )
    return pl.pallas_call(
        _ffn_body,
        grid_spec=grid_spec,
        out_shape=jax.ShapeDtypeStruct((P, D), jnp.float32),
        compiler_params=pltpu.CompilerParams(
            dimension_semantics=("arbitrary",)),
    )(step_expert, active, xs, W1, W3, W2)


@functools.partial(
    pl.kernel,
    mesh=_sc_mesh,
    out_type=jax.ShapeDtypeStruct((T, D), jnp.float32),
    scratch_types=[
        pltpu.VMEM((TW,), jnp.int32),
        pltpu.VMEM((TW,), jnp.int32),
        pltpu.VMEM((TW,), jnp.float32),
        pltpu.VMEM((TW,), jnp.float32),
        pltpu.VMEM((TW, D), jnp.float32),
        pltpu.VMEM((TW, D), jnp.float32),
        pltpu.SemaphoreType.DMA,
    ],
)
def _sc_combine(ys_hbm, p0_hbm, p1_hbm, wa_hbm, wb_hbm, out_hbm,
                i0, i1, wv0, wv1, b0, b1, sem):
    wid = lax.axis_index("s") * NC + lax.axis_index("c")
    base = wid * TW
    pltpu.sync_copy(p0_hbm.at[pl.ds(base, TW)], i0)
    pltpu.sync_copy(p1_hbm.at[pl.ds(base, TW)], i1)
    pltpu.sync_copy(wa_hbm.at[pl.ds(base, TW)], wv0)
    pltpu.sync_copy(wb_hbm.at[pl.ds(base, TW)], wv1)
    c0 = pltpu.async_copy(ys_hbm.at[i0], b0, sem)
    c1 = pltpu.async_copy(ys_hbm.at[i1], b1, sem)
    c0.wait()
    c1.wait()

    dnums = lax.GatherDimensionNumbers(
        offset_dims=(), collapsed_slice_dims=(0,), start_index_map=(0,))

    def _bcast(vec, lane):
        idx = jnp.full((16, 1), lane, jnp.int32)
        return lax.gather(vec, idx, dnums, (1,),
                          mode=lax.GatherScatterMode.PROMISE_IN_BOUNDS)

    def _comb_row(r, carry):
        g = r // 16
        lane = r - g * 16
        wa_b = _bcast(wv0[pl.ds(g * 16, 16)], lane)
        wb_b = _bcast(wv1[pl.ds(g * 16, 16)], lane)
        for j in range(D // 16):
            sl = pl.ds(j * 16, 16)
            b0[r, sl] = wa_b * b0[r, sl] + wb_b * b1[r, sl]
        return carry

    lax.fori_loop(0, TW, _comb_row, None)
    pltpu.sync_copy(b0, out_hbm.at[pl.ds(base, TW)])


def kernel(x, router_logits, W1, W2, W3):
    step_expert, active, wa, wb, p0, p1 = _routing_tables(router_logits)
    xs = _sc_dispatch(x, p0, p1)
    ys = _grouped_ffn(xs, W1, W2, W3, step_expert, active)
    return _sc_combine(ys, p0, p1, wa, wb)


# speedup vs baseline: 1.3735x; 1.0241x over previous
import functools

import jax
import jax.numpy as jnp
from jax import lax
from jax.experimental import pallas as pl
from jax.experimental.pallas import tpu as pltpu
from jax.experimental.pallas import tpu_sc as plsc

T, D, F, E, K = 2048, 768, 2048, 8, 2
TM = 256
NSTEPS = 24
P = NSTEPS * TM

NC, NS = 2, 16
NW = NC * NS
TW = T // NW

_sc_mesh = plsc.VectorSubcoreMesh(core_axis_name="c", subcore_axis_name="s")


def _routing_tables(router_logits):
    eids = jnp.arange(E, dtype=jnp.int32)
    m1 = jnp.max(router_logits, axis=-1)
    a1 = jnp.argmax(router_logits, axis=-1).astype(jnp.int32)
    masked = jnp.where(eids[None, :] == a1[:, None], -jnp.inf, router_logits)
    m2 = jnp.max(masked, axis=-1)
    a2 = jnp.argmax(masked, axis=-1).astype(jnp.int32)
    w1 = jax.nn.sigmoid(m1 - m2)
    flat_e = jnp.concatenate([a1, a2])
    S = T * K
    onehot = (flat_e[:, None] == eids[None, :]).astype(jnp.int32)
    csum = jnp.cumsum(onehot, axis=0)
    counts = csum[-1]
    rank = jnp.sum(csum * onehot, axis=1) - 1
    tiles_e = (counts + TM - 1) // TM
    tile_cum = jnp.cumsum(tiles_e)
    pad_off = (tile_cum - tiles_e) * TM
    ppos = pad_off[flat_e] + rank
    step_ids = jnp.arange(NSTEPS, dtype=jnp.int32)
    step_expert = jnp.sum(
        (tile_cum[None, :] <= step_ids[:, None]).astype(jnp.int32), axis=1)
    last_used = jnp.max(jnp.where(counts > 0, jnp.arange(E, dtype=jnp.int32), 0))
    step_expert = jnp.minimum(step_expert, last_used)
    active = (step_ids < tile_cum[-1]).astype(jnp.int32)
    pp = ppos.reshape(K, T)
    return (step_expert, active, w1, 1.0 - w1,
            pp[0].astype(jnp.int32), pp[1].astype(jnp.int32))


@functools.partial(
    pl.kernel,
    mesh=_sc_mesh,
    out_type=jax.ShapeDtypeStruct((P, D), jnp.float32),
    scratch_types=[
        pltpu.VMEM((TW,), jnp.int32),
        pltpu.VMEM((TW,), jnp.int32),
        pltpu.VMEM((TW, D), jnp.float32),
        pltpu.SemaphoreType.DMA,
    ],
)
def _sc_dispatch(x_hbm, p0_hbm, p1_hbm, xs_hbm, i0, i1, buf, sem):
    wid = lax.axis_index("s") * NC + lax.axis_index("c")
    base = wid * TW
    pltpu.sync_copy(p0_hbm.at[pl.ds(base, TW)], i0)
    pltpu.sync_copy(p1_hbm.at[pl.ds(base, TW)], i1)
    pltpu.sync_copy(x_hbm.at[pl.ds(base, TW)], buf)
    s0 = pltpu.async_copy(buf, xs_hbm.at[i0], sem)
    s1 = pltpu.async_copy(buf, xs_hbm.at[i1], sem)
    s0.wait()
    s1.wait()


def _ffn_body(se_ref, act_ref, xs_ref, w1_ref, w3_ref, w2_ref, ys_ref):
    s = pl.program_id(0)

    @pl.when(act_ref[s] > 0)
    def _():
        xt = xs_ref[...]
        g = jax.lax.dot_general(xt, w1_ref[0], (((1,), (1,)), ((), ())),
                                preferred_element_type=jnp.float32)
        u = jax.lax.dot_general(xt, w3_ref[0], (((1,), (1,)), ((), ())),
                                preferred_element_type=jnp.float32)
        h = (g * jax.nn.sigmoid(g)) * u
        ys_ref[...] = jax.lax.dot_general(
            h, w2_ref[0], (((1,), (1,)), ((), ())),
            preferred_element_type=jnp.float32)


def _grouped_ffn(xs, W1, W2, W3, step_expert, active):
    grid_spec = pltpu.PrefetchScalarGridSpec(
        num_scalar_prefetch=2,
        grid=(NSTEPS,),
        in_specs=[
            pl.BlockSpec((TM, D), lambda s, se, act: (s, 0)),
            pl.BlockSpec((1, F, D), lambda s, se, act: (se[s], 0, 0)),
            pl.BlockSpec((1, F, D), lambda s, se, act: (se[s], 0, 0)),
            pl.BlockSpec((1, D, F), lambda s, se, act: (se[s], 0, 0)),
        ],
        out_specs=pl.BlockSpec((TM, D), lambda s, se, act: (s, 0)),
    )
    return pl.pallas_call(
        _ffn_body,
        grid_spec=grid_spec,
        out_shape=jax.ShapeDtypeStruct((P, D), jnp.float32),
        compiler_params=pltpu.CompilerParams(
            dimension_semantics=("arbitrary",)),
    )(step_expert, active, xs, W1, W3, W2)


@functools.partial(
    pl.kernel,
    mesh=_sc_mesh,
    out_type=jax.ShapeDtypeStruct((T, D), jnp.float32),
    scratch_types=[
        pltpu.VMEM((2, TW // 2), jnp.int32),
        pltpu.VMEM((2, TW // 2), jnp.int32),
        pltpu.VMEM((TW,), jnp.float32),
        pltpu.VMEM((TW,), jnp.float32),
        [pltpu.VMEM((TW // 2, D), jnp.float32)] * 2,
        [pltpu.VMEM((TW // 2, D), jnp.float32)] * 2,
        [pltpu.SemaphoreType.DMA] * 2,
        pltpu.SemaphoreType.DMA,
    ],
)
def _sc_combine(ys_hbm, p0_hbm, p1_hbm, wa_hbm, wb_hbm, out_hbm,
                i0, i1, wv0, wv1, b0s, b1s, gsems, wsem):
    wid = lax.axis_index("s") * NC + lax.axis_index("c")
    base = wid * TW
    hw = TW // 2
    pltpu.sync_copy(p0_hbm.at[wid], i0)
    pltpu.sync_copy(p1_hbm.at[wid], i1)
    pltpu.sync_copy(wa_hbm.at[pl.ds(base, TW)], wv0)
    pltpu.sync_copy(wb_hbm.at[pl.ds(base, TW)], wv1)
    gathers = []
    for c in range(2):
        gathers.append((
            pltpu.async_copy(ys_hbm.at[i0.at[c]], b0s[c], gsems[c]),
            pltpu.async_copy(ys_hbm.at[i1.at[c]], b1s[c], gsems[c]),
        ))

    dnums = lax.GatherDimensionNumbers(
        offset_dims=(), collapsed_slice_dims=(0,), start_index_map=(0,))

    def _bcast(vec, lane):
        idx = jnp.full((16, 1), lane, jnp.int32)
        return lax.gather(vec, idx, dnums, (1,),
                          mode=lax.GatherScatterMode.PROMISE_IN_BOUNDS)

    writes = []
    for c in range(2):
        gathers[c][0].wait()
        gathers[c][1].wait()
        b0, b1 = b0s[c], b1s[c]

        def _comb_row(r, carry):
            row = c * hw + r
            g = row // 16
            lane = row - g * 16
            wa_b = _bcast(wv0[pl.ds(g * 16, 16)], lane)
            wb_b = _bcast(wv1[pl.ds(g * 16, 16)], lane)
            for j in range(D // 16):
                sl = pl.ds(j * 16, 16)
                b0[r, sl] = wa_b * b0[r, sl] + wb_b * b1[r, sl]
            return carry

        lax.fori_loop(0, hw, _comb_row, None)
        writes.append(pltpu.async_copy(
            b0, out_hbm.at[pl.ds(base + c * hw, hw)], wsem))
    writes[0].wait()
    writes[1].wait()


def kernel(x, router_logits, W1, W2, W3):
    step_expert, active, wa, wb, p0, p1 = _routing_tables(router_logits)
    xs = _sc_dispatch(x, p0, p1)
    ys = _grouped_ffn(xs, W1, W2, W3, step_expert, active)
    return _sc_combine(ys, p0.reshape(NW, 2, TW // 2), p1.reshape(NW, 2, TW // 2),
                       wa, wb)
